# baseline (speedup 1.0000x reference)
import functools

import jax
import jax.numpy as jnp
from jax import lax
from jax.experimental import pallas as pl
from jax.experimental.pallas import tpu as pltpu
from jax.experimental.pallas import tpu_sc as plsc

_NFIELD = 26
_NEMB = 16
_BATCH = 16384
_TOTAL = _BATCH * _NFIELD

_NC = 2
_NS = 16
_NW = _NC * _NS
_PER_W = _TOTAL // _NW
_IDXROW = 128
_ROWS_PER_W = _PER_W // _IDXROW
_CH_IDXROWS = 13
_CH = _CH_IDXROWS * _IDXROW
_NCHUNK = _PER_W // _CH


def _sc_body(idx_hbm, val_hbm, table_hbm, out_hbm, idx_v, val_v, rows_v, sem):
    c = lax.axis_index("c")
    s = lax.axis_index("s")
    wid = s * _NC + c
    base = wid * _PER_W
    irow0 = wid * _ROWS_PER_W

    pltpu.sync_copy(idx_hbm.at[pl.ds(irow0, _ROWS_PER_W)], idx_v)
    pltpu.sync_copy(val_hbm.at[pl.ds(base, _PER_W)], val_v)

    def chunk_body(g, _):
        copies = []
        for k in range(_CH_IDXROWS):
            copies.append(
                pltpu.async_copy(
                    table_hbm.at[idx_v.at[g * _CH_IDXROWS + k]],
                    rows_v.at[pl.ds(k * _IDXROW, _IDXROW)],
                    sem,
                )
            )
        for cp in copies:
            cp.wait()

        voff = g * _CH

        dnums = lax.GatherDimensionNumbers(
            offset_dims=(), collapsed_slice_dims=(0,), start_index_map=(0,)
        )

        def mul_body(j, _):
            r0 = j * _NEMB
            vals = val_v[pl.ds(voff + r0, 16)]
            for i in range(_NEMB):
                r = r0 + i
                b = lax.gather(
                    vals,
                    jnp.full((16, 1), i, jnp.int32),
                    dnums,
                    slice_sizes=(1,),
                    mode=lax.GatherScatterMode.PROMISE_IN_BOUNDS,
                )
                rows_v[r, :] = rows_v[r, :] * b
            return 0

        lax.fori_loop(0, _CH // _NEMB, mul_body, 0, unroll=False)
        pltpu.sync_copy(rows_v, out_hbm.at[pl.ds(base + voff, _CH)])
        return 0

    lax.fori_loop(0, _NCHUNK, chunk_body, 0, unroll=False)


@jax.jit
def kernel(idx, value, table):
    idx2d = idx.reshape(_TOTAL // _IDXROW, _IDXROW)
    val = value.reshape(_TOTAL)
    mesh = plsc.VectorSubcoreMesh(core_axis_name="c", subcore_axis_name="s")
    out = pl.kernel(
        _sc_body,
        out_type=jax.ShapeDtypeStruct((_TOTAL, _NEMB), jnp.float32),
        mesh=mesh,
        compiler_params=pltpu.CompilerParams(use_tc_tiling_on_sc=False),
        scratch_types=[
            pltpu.VMEM((_ROWS_PER_W, _IDXROW), jnp.int32),
            pltpu.VMEM((_PER_W,), jnp.float32),
            pltpu.VMEM((_CH, _NEMB), jnp.float32),
            pltpu.SemaphoreType.DMA,
        ],
    )(idx2d, val, table)
    return out.reshape(_BATCH, _NFIELD, _NEMB)


# speedup vs baseline: 1.4846x; 1.4846x over previous
import functools

import jax
import jax.numpy as jnp
from jax import lax
from jax.experimental import pallas as pl
from jax.experimental.pallas import tpu as pltpu
from jax.experimental.pallas import tpu_sc as plsc

_NFIELD = 26
_NEMB = 16
_BATCH = 16384

_NW = 32
_BW = _BATCH // _NW
_NGRP = _BW // _NEMB


def _gather_body(idxT_hbm, valT_hbm, table_hbm, out_hbm,
                 idx_v, val_v, rows_v, col_v, sem):
    c = lax.axis_index("c")
    s = lax.axis_index("s")
    wid = s * 2 + c
    b0 = wid * _BW

    pltpu.sync_copy(idxT_hbm.at[:, pl.ds(b0, _BW)], idx_v)
    pltpu.sync_copy(valT_hbm.at[:, pl.ds(b0, _BW)], val_v)

    iota16 = lax.broadcasted_iota(jnp.int32, (16,), 0)

    for f in range(_NFIELD):
        copies = [
            pltpu.async_copy(
                table_hbm.at[idx_v.at[f, pl.ds(k * 128, 128)]],
                rows_v.at[pl.ds(k * 128, 128)],
                sem,
            )
            for k in range(_BW // 128)
        ]
        for cp in copies:
            cp.wait()

        def grp_body(j, _, f=f):
            r0 = j * _NEMB
            vals = val_v[f, pl.ds(r0, _NEMB)]
            ridx = iota16 + r0
            for e in range(_NEMB):
                col = plsc.load_gather(rows_v, [ridx, jnp.full((16,), e, jnp.int32)])
                col_v[e, pl.ds(r0, _NEMB)] = col * vals
            return 0

        lax.fori_loop(0, _NGRP, grp_body, 0, unroll=False)
        pltpu.sync_copy(col_v, out_hbm.at[f, :, pl.ds(b0, _BW)])


@jax.jit
def kernel(idx, value, table):
    idxT = idx.T
    valT = value.T
    mesh = plsc.VectorSubcoreMesh(core_axis_name="c", subcore_axis_name="s")
    out = pl.kernel(
        _gather_body,
        out_type=jax.ShapeDtypeStruct((_NFIELD, _NEMB, _BATCH), jnp.float32),
        mesh=mesh,
        scratch_types=[
            pltpu.VMEM((_NFIELD, _BW), jnp.int32),
            pltpu.VMEM((_NFIELD, _BW), jnp.float32),
            pltpu.VMEM((_BW, _NEMB), jnp.float32),
            pltpu.VMEM((_NEMB, _BW), jnp.float32),
            pltpu.SemaphoreType.DMA,
        ],
        compiler_params=pltpu.CompilerParams(
            use_tc_tiling_on_sc=False, needs_layout_passes=False
        ),
    )(idxT, valT, table)
    return out.transpose(2, 0, 1)


# speedup vs baseline: 1.5876x; 1.0694x over previous
import functools

import jax
import jax.numpy as jnp
from jax import lax
from jax.experimental import pallas as pl
from jax.experimental.pallas import tpu as pltpu
from jax.experimental.pallas import tpu_sc as plsc

_NFIELD = 26
_NEMB = 16
_BATCH = 16384

_NW = 32
_BW = _BATCH // _NW
_NGRP = _BW // _NEMB


def _gather_body(idxT_hbm, valT_hbm, table_hbm, out_hbm,
                 idx_v, val_v, rows_v, col_v, sem_g, sem_o):
    c = lax.axis_index("c")
    s = lax.axis_index("s")
    wid = s * 2 + c
    b0 = wid * _BW

    pltpu.sync_copy(idxT_hbm.at[:, pl.ds(b0, _BW)], idx_v)
    pltpu.sync_copy(valT_hbm.at[:, pl.ds(b0, _BW)], val_v)

    iota16 = lax.broadcasted_iota(jnp.int32, (16,), 0)

    def fire_gather(f):
        b = f % 2
        return [
            pltpu.async_copy(
                table_hbm.at[idx_v.at[f, pl.ds(k * 128, 128)]],
                rows_v.at[b, pl.ds(k * 128, 128)],
                sem_g.at[b],
            )
            for k in range(_BW // 128)
        ]

    gd = {0: fire_gather(0)}
    od = {}
    for f in range(_NFIELD):
        b = f % 2
        if f + 1 < _NFIELD:
            gd[f + 1] = fire_gather(f + 1)
        for d in gd[f]:
            d.wait()
        if f >= 2:
            od[f - 2].wait()

        rows_b = rows_v.at[b]
        col_b = col_v.at[b]

        def grp_body(j, _, f=f, rows_b=rows_b, col_b=col_b):
            r0 = j * _NEMB
            vals = val_v[f, pl.ds(r0, _NEMB)]
            ridx = iota16 + r0
            for e in range(_NEMB):
                col = plsc.load_gather(rows_b, [ridx, jnp.full((16,), e, jnp.int32)])
                col_b[e, pl.ds(r0, _NEMB)] = col * vals
            return 0

        lax.fori_loop(0, _NGRP, grp_body, 0, unroll=False)
        od[f] = pltpu.async_copy(
            col_b, out_hbm.at[f, :, pl.ds(b0, _BW)], sem_o.at[b]
        )
    od[_NFIELD - 2].wait()
    od[_NFIELD - 1].wait()


@jax.jit
def kernel(idx, value, table):
    idxT = idx.T
    valT = value.T
    mesh = plsc.VectorSubcoreMesh(core_axis_name="c", subcore_axis_name="s")
    out = pl.kernel(
        _gather_body,
        out_type=jax.ShapeDtypeStruct((_NFIELD, _NEMB, _BATCH), jnp.float32),
        mesh=mesh,
        scratch_types=[
            pltpu.VMEM((_NFIELD, _BW), jnp.int32),
            pltpu.VMEM((_NFIELD, _BW), jnp.float32),
            pltpu.VMEM((2, _BW, _NEMB), jnp.float32),
            pltpu.VMEM((2, _NEMB, _BW), jnp.float32),
            pltpu.SemaphoreType.DMA((2,)),
            pltpu.SemaphoreType.DMA((2,)),
        ],
        compiler_params=pltpu.CompilerParams(
            use_tc_tiling_on_sc=False, needs_layout_passes=False
        ),
    )(idxT, valT, table)
    return out.transpose(2, 0, 1)
